# baseline (speedup 1.0000x reference)
import functools

import jax
import jax.numpy as jnp
from jax import lax
from jax.experimental import pallas as pl
from jax.experimental.pallas import tpu as pltpu
from jax.experimental.pallas import tpu_sc as plsc

B = 16
L = 16
NC, NS = 2, 16
NW = NC * NS


def _lane(vec, s, iota):
    return jnp.sum(jnp.where(iota == s, vec, 0))


def _seg_bounds(bidx_v, r, iota):
    lo = jnp.zeros((L,), jnp.int32)
    hi = jnp.full((L,), r, jnp.int32)
    steps = max(1, (r).bit_length())

    def body(_, carry):
        lo, hi = carry
        active = lo < hi
        mid = lax.div(lo + hi, 2)
        vals = plsc.load_gather(bidx_v, [jnp.minimum(mid, r - 1)])
        pred = vals < iota
        lo = jnp.where(active & pred, mid + 1, lo)
        hi = jnp.where(active & jnp.logical_not(pred), mid, hi)
        return lo, hi

    lo, hi = lax.fori_loop(0, steps, body, (lo, hi))
    starts = [_lane(lo, s, iota) for s in range(B)]
    ends = starts[1:] + [jnp.int32(r)]
    return starts, ends


def _make_sums_kernel(c, r, ch):
    nch = r // ch
    mesh = plsc.VectorSubcoreMesh(core_axis_name="c", subcore_axis_name="s")

    @functools.partial(
        pl.kernel,
        out_type=[
            jax.ShapeDtypeStruct((NW, B * c), jnp.float32),
            jax.ShapeDtypeStruct((NW * L,), jnp.float32),
        ],
        mesh=mesh,
        compiler_params=pltpu.CompilerParams(needs_layout_passes=False),
        scratch_types=[
            pltpu.VMEM((r,), jnp.int32),
            pltpu.VMEM((ch * c,), jnp.float32),
            pltpu.VMEM((ch * c,), jnp.float32),
            pltpu.VMEM((ch * c,), jnp.float32),
            pltpu.VMEM((ch * c,), jnp.float32),
            pltpu.VMEM((B * c,), jnp.float32),
            pltpu.VMEM((L,), jnp.float32),
            pltpu.SemaphoreType.DMA,
            pltpu.SemaphoreType.DMA,
            pltpu.SemaphoreType.DMA,
            pltpu.SemaphoreType.DMA,
            pltpu.SemaphoreType.DMA,
        ],
    )
    def sums_kernel(feat_hbm, bidx_hbm, psum_hbm, pcnt_hbm,
                    bidx_v, f0, f1, f2, f3, acc_v, cnt_v,
                    semi, sem0, sem1, sem2, sem3):
        wid = lax.axis_index("c") * NS + lax.axis_index("s")
        base = wid * r
        iota = lax.iota(jnp.int32, L)
        zero16 = jnp.zeros((L,), jnp.float32)

        fbs = (f0, f1, f2, f3)
        sems = (sem0, sem1, sem2, sem3)
        pltpu.async_copy(bidx_hbm.at[pl.ds(base, r)], bidx_v, semi)
        for b in range(4):
            pltpu.async_copy(
                feat_hbm.at[pl.ds((base + b * ch) * c, ch * c)], fbs[b],
                sems[b])

        for q in range(B * c // L):
            acc_v[pl.ds(q * L, L)] = zero16

        pltpu.make_async_copy(bidx_hbm.at[pl.ds(base, r)], bidx_v, semi).wait()
        starts, ends = _seg_bounds(bidx_v, r, iota)

        cnt_f = zero16
        for s in range(B):
            cnt_f = jnp.where(iota == s,
                              (ends[s] - starts[s]).astype(jnp.float32), cnt_f)
        cnt_v[...] = cnt_f

        def chunk_pair(p, carry):
            for b in range(4):
                j = p * 4 + b
                cb = j * ch
                fb = fbs[b]
                semb = sems[b]
                pltpu.make_async_copy(
                    feat_hbm.at[pl.ds((base + cb) * c, ch * c)], fb, semb
                ).wait()
                for s in range(B):
                    lo = jnp.maximum(starts[s] - cb, 0)
                    hi = jnp.minimum(ends[s] - cb, ch)

                    @pl.when(hi > lo)
                    def _(s=s, lo=lo, hi=hi, fb=fb):
                        acc8 = tuple(
                            acc_v[pl.ds(s * c + k * L, L)] for k in range(c // L)
                        )

                        @plsc.parallel_loop(lo, hi, unroll=2, carry=acc8)
                        def a8(rr, a8, fb=fb):
                            return tuple(
                                a + fb[pl.ds(rr * c + k * L, L)]
                                for k, a in enumerate(a8)
                            )

                        for k in range(c // L):
                            acc_v[pl.ds(s * c + k * L, L)] = a8[k]

                nxt = j + 4

                @pl.when(nxt < nch)
                def _(nxt=nxt, fb=fb, semb=semb):
                    pltpu.async_copy(
                        feat_hbm.at[pl.ds((base + nxt * ch) * c, ch * c)],
                        fb, semb)
            return carry

        lax.fori_loop(0, nch // 4, chunk_pair, 0)

        pltpu.sync_copy(acc_v, psum_hbm.at[wid])
        pltpu.sync_copy(cnt_v, pcnt_hbm.at[pl.ds(wid * L, L)])

    return sums_kernel


def _p1(bidx_ref, feat_ref, sums_ref, cnt_ref):
    i = pl.program_id(0)
    rb = feat_ref.shape[0]
    b = bidx_ref[0, 0, :]
    onehot = (b[:, None] == jax.lax.broadcasted_iota(jnp.int32, (rb, B), 1)
              ).astype(jnp.float32)
    part = jax.lax.dot_general(onehot, feat_ref[...],
                               (((0,), (0,)), ((), ())),
                               preferred_element_type=jnp.float32)
    pcnt = jnp.sum(onehot, axis=0)[:, None]

    @pl.when(i == 0)
    def _():
        sums_ref[...] = jnp.zeros_like(sums_ref)
        cnt_ref[...] = jnp.zeros_like(cnt_ref)

    sums_ref[...] += part
    cnt_ref[...] += jnp.broadcast_to(pcnt, cnt_ref.shape)


def _p3g(bidx_ref, feat_ref, tsums_ref, tcnt_ref, psum_ref, pcnt_ref, w_ref,
         out_ref, gates_ref):
    i = pl.program_id(0)
    rb = feat_ref.shape[0]

    @pl.when(i == 0)
    def _():
        sums = tsums_ref[...] + jnp.sum(psum_ref[...], axis=0)
        cnt = tcnt_ref[...] + jnp.sum(pcnt_ref[...], axis=0)[:, None]
        m = sums / jnp.maximum(cnt, 1.0)
        w0 = w_ref[0, 0]
        w1 = w_ref[0, 1]
        w2 = w_ref[0, 2]
        zero = jnp.zeros((m.shape[0], 1), jnp.float32)
        left = jnp.concatenate([zero, m[:, :-1]], axis=1)
        right = jnp.concatenate([m[:, 1:], zero], axis=1)
        y = w0 * left + w1 * m + w2 * right
        gates_ref[...] = jax.nn.sigmoid(y)

    b = bidx_ref[0, 0, :]
    onehot = (b[:, None] == jax.lax.broadcasted_iota(jnp.int32, (rb, B), 1)
              ).astype(jnp.float32)
    g = jax.lax.dot_general(onehot, gates_ref[...],
                            (((1,), (0,)), ((), ())),
                            preferred_element_type=jnp.float32)
    out_ref[...] = feat_ref[...] * g


N_SC_FRAC_NUM, N_SC_FRAC_DEN = 2, 5


def kernel(features, batch_idx, W):
    n, c = features.shape
    assert c % L == 0
    rb = 8000
    grain = NW * 1000
    n_sc = max(grain, (n * N_SC_FRAC_NUM // N_SC_FRAC_DEN) // grain * grain)
    r = n_sc // NW
    ch_a = 200
    assert r % ch_a == 0 and (r // ch_a) % 4 == 0 and r % 8 == 0, (r, ch_a)
    n_tc = n - n_sc

    featflat = features.reshape(-1)

    psum, pcnt = _make_sums_kernel(c, r, ch_a)(featflat, batch_idx)

    assert n_tc % rb == 0 and n_sc % rb == 0
    nb_tc = n_tc // rb
    blk0 = n_sc // rb
    bidx3_tc = batch_idx[n_sc:].reshape(nb_tc, 1, rb)
    tsums, tcnt = pl.pallas_call(
        _p1,
        grid=(nb_tc,),
        in_specs=[
            pl.BlockSpec((1, 1, rb), lambda i: (i, 0, 0)),
            pl.BlockSpec((rb, c), lambda i: (i + blk0, 0)),
        ],
        out_specs=[
            pl.BlockSpec((B, c), lambda i: (0, 0)),
            pl.BlockSpec((B, c), lambda i: (0, 0)),
        ],
        out_shape=[
            jax.ShapeDtypeStruct((B, c), jnp.float32),
            jax.ShapeDtypeStruct((B, c), jnp.float32),
        ],
    )(bidx3_tc, features)

    rb3 = 20000
    assert n % rb3 == 0
    nb = n // rb3
    bidx3 = batch_idx.reshape(nb, 1, rb3)
    out = pl.pallas_call(
        _p3g,
        grid=(nb,),
        in_specs=[
            pl.BlockSpec((1, 1, rb3), lambda i: (i, 0, 0)),
            pl.BlockSpec((rb3, c), lambda i: (i, 0)),
            pl.BlockSpec((B, c), lambda i: (0, 0)),
            pl.BlockSpec((B, c), lambda i: (0, 0)),
            pl.BlockSpec((NW, B, c), lambda i: (0, 0, 0)),
            pl.BlockSpec((NW, L), lambda i: (0, 0)),
            pl.BlockSpec((1, 3), lambda i: (0, 0)),
        ],
        out_specs=pl.BlockSpec((rb3, c), lambda i: (i, 0)),
        out_shape=jax.ShapeDtypeStruct((n, c), jnp.float32),
        scratch_shapes=[pltpu.VMEM((B, c), jnp.float32)],
    )(bidx3, features, tsums, tcnt, psum.reshape(NW, B, c),
      pcnt.reshape(NW, L), W.reshape(1, 3))
    return out


# speedup vs baseline: 1.0114x; 1.0114x over previous
import functools

import jax
import jax.numpy as jnp
from jax import lax
from jax.experimental import pallas as pl
from jax.experimental.pallas import tpu as pltpu
from jax.experimental.pallas import tpu_sc as plsc

B = 16
L = 16
NC, NS = 2, 16
NW = NC * NS


def _lane(vec, s, iota):
    return jnp.sum(jnp.where(iota == s, vec, 0))


def _seg_bounds(bidx_v, r, iota):
    lo = jnp.zeros((L,), jnp.int32)
    hi = jnp.full((L,), r, jnp.int32)
    steps = max(1, (r).bit_length())

    def body(_, carry):
        lo, hi = carry
        active = lo < hi
        mid = lax.div(lo + hi, 2)
        vals = plsc.load_gather(bidx_v, [jnp.minimum(mid, r - 1)])
        pred = vals < iota
        lo = jnp.where(active & pred, mid + 1, lo)
        hi = jnp.where(active & jnp.logical_not(pred), mid, hi)
        return lo, hi

    lo, hi = lax.fori_loop(0, steps, body, (lo, hi))
    starts = [_lane(lo, s, iota) for s in range(B)]
    ends = starts[1:] + [jnp.int32(r)]
    return starts, ends


def _make_sums_kernel(c, r, ch):
    nch = r // ch
    mesh = plsc.VectorSubcoreMesh(core_axis_name="c", subcore_axis_name="s")

    @functools.partial(
        pl.kernel,
        out_type=[
            jax.ShapeDtypeStruct((NW, B * c), jnp.float32),
            jax.ShapeDtypeStruct((NW * L,), jnp.float32),
        ],
        mesh=mesh,
        compiler_params=pltpu.CompilerParams(needs_layout_passes=False),
        scratch_types=[
            pltpu.VMEM((r,), jnp.int32),
            pltpu.VMEM((ch * c,), jnp.float32),
            pltpu.VMEM((ch * c,), jnp.float32),
            pltpu.VMEM((B * c,), jnp.float32),
            pltpu.VMEM((L,), jnp.float32),
            pltpu.SemaphoreType.DMA,
            pltpu.SemaphoreType.DMA,
            pltpu.SemaphoreType.DMA,
        ],
    )
    def sums_kernel(feat_hbm, bidx_hbm, psum_hbm, pcnt_hbm,
                    bidx_v, f0, f1, acc_v, cnt_v, semi, sem0, sem1):
        wid = lax.axis_index("c") * NS + lax.axis_index("s")
        base = wid * r
        iota = lax.iota(jnp.int32, L)
        zero16 = jnp.zeros((L,), jnp.float32)

        fbs = (f0, f1)
        sems = (sem0, sem1)
        pltpu.async_copy(bidx_hbm.at[pl.ds(base, r)], bidx_v, semi)
        for b in range(2):
            pltpu.async_copy(
                feat_hbm.at[pl.ds((base + b * ch) * c, ch * c)], fbs[b],
                sems[b])

        for q in range(B * c // L):
            acc_v[pl.ds(q * L, L)] = zero16

        pltpu.make_async_copy(bidx_hbm.at[pl.ds(base, r)], bidx_v, semi).wait()
        starts, ends = _seg_bounds(bidx_v, r, iota)

        cnt_f = zero16
        for s in range(B):
            cnt_f = jnp.where(iota == s,
                              (ends[s] - starts[s]).astype(jnp.float32), cnt_f)
        cnt_v[...] = cnt_f

        def chunk_pair(p, carry):
            for b in range(2):
                j = p * 2 + b
                cb = j * ch
                fb = fbs[b]
                semb = sems[b]
                pltpu.make_async_copy(
                    feat_hbm.at[pl.ds((base + cb) * c, ch * c)], fb, semb
                ).wait()
                for s in range(B):
                    lo = jnp.maximum(starts[s] - cb, 0)
                    hi = jnp.minimum(ends[s] - cb, ch)

                    @pl.when(hi > lo)
                    def _(s=s, lo=lo, hi=hi, fb=fb):
                        acc8 = tuple(
                            acc_v[pl.ds(s * c + k * L, L)] for k in range(c // L)
                        )

                        @plsc.parallel_loop(lo, hi, unroll=2, carry=acc8)
                        def a8(rr, a8, fb=fb):
                            return tuple(
                                a + fb[pl.ds(rr * c + k * L, L)]
                                for k, a in enumerate(a8)
                            )

                        for k in range(c // L):
                            acc_v[pl.ds(s * c + k * L, L)] = a8[k]

                nxt = j + 2

                @pl.when(nxt < nch)
                def _(nxt=nxt, fb=fb, semb=semb):
                    pltpu.async_copy(
                        feat_hbm.at[pl.ds((base + nxt * ch) * c, ch * c)],
                        fb, semb)
            return carry

        lax.fori_loop(0, nch // 2, chunk_pair, 0)

        pltpu.sync_copy(acc_v, psum_hbm.at[wid])
        pltpu.sync_copy(cnt_v, pcnt_hbm.at[pl.ds(wid * L, L)])

    return sums_kernel


def _p1(bidx_ref, feat_ref, sums_ref, cnt_ref):
    i = pl.program_id(0)
    rb = feat_ref.shape[0]
    b = bidx_ref[0, 0, :]
    onehot = (b[:, None] == jax.lax.broadcasted_iota(jnp.int32, (rb, B), 1)
              ).astype(jnp.float32)
    part = jax.lax.dot_general(onehot, feat_ref[...],
                               (((0,), (0,)), ((), ())),
                               preferred_element_type=jnp.float32)
    pcnt = jnp.sum(onehot, axis=0)[:, None]

    @pl.when(i == 0)
    def _():
        sums_ref[...] = jnp.zeros_like(sums_ref)
        cnt_ref[...] = jnp.zeros_like(cnt_ref)

    sums_ref[...] += part
    cnt_ref[...] += jnp.broadcast_to(pcnt, cnt_ref.shape)


def _p3g(bidx_ref, feat_ref, tsums_ref, tcnt_ref, psum_ref, pcnt_ref, w_ref,
         out_ref, gates_ref):
    i = pl.program_id(0)
    rb = feat_ref.shape[0]

    @pl.when(i == 0)
    def _():
        sums = tsums_ref[...] + jnp.sum(psum_ref[...], axis=0)
        cnt = tcnt_ref[...] + jnp.sum(pcnt_ref[...], axis=0)[:, None]
        m = sums / jnp.maximum(cnt, 1.0)
        w0 = w_ref[0, 0]
        w1 = w_ref[0, 1]
        w2 = w_ref[0, 2]
        zero = jnp.zeros((m.shape[0], 1), jnp.float32)
        left = jnp.concatenate([zero, m[:, :-1]], axis=1)
        right = jnp.concatenate([m[:, 1:], zero], axis=1)
        y = w0 * left + w1 * m + w2 * right
        gates_ref[...] = jax.nn.sigmoid(y)

    b = bidx_ref[0, 0, :]
    onehot = (b[:, None] == jax.lax.broadcasted_iota(jnp.int32, (rb, B), 1)
              ).astype(jnp.float32)
    g = jax.lax.dot_general(onehot, gates_ref[...],
                            (((1,), (0,)), ((), ())),
                            preferred_element_type=jnp.float32)
    out_ref[...] = feat_ref[...] * g


N_SC_FRAC_NUM, N_SC_FRAC_DEN = 2, 5


def kernel(features, batch_idx, W):
    n, c = features.shape
    assert c % L == 0
    rb = 8000
    grain = NW * 1000
    n_sc = max(grain, (n * N_SC_FRAC_NUM // N_SC_FRAC_DEN) // grain * grain)
    r = n_sc // NW
    ch_a = 400
    assert r % ch_a == 0 and (r // ch_a) % 2 == 0 and r % 8 == 0, (r, ch_a)
    n_tc = n - n_sc

    featflat = features.reshape(-1)

    psum, pcnt = _make_sums_kernel(c, r, ch_a)(featflat, batch_idx)

    assert n_tc % rb == 0 and n_sc % rb == 0
    nb_tc = n_tc // rb
    blk0 = n_sc // rb
    bidx3_tc = batch_idx[n_sc:].reshape(nb_tc, 1, rb)
    tsums, tcnt = pl.pallas_call(
        _p1,
        grid=(nb_tc,),
        in_specs=[
            pl.BlockSpec((1, 1, rb), lambda i: (i, 0, 0)),
            pl.BlockSpec((rb, c), lambda i: (i + blk0, 0)),
        ],
        out_specs=[
            pl.BlockSpec((B, c), lambda i: (0, 0)),
            pl.BlockSpec((B, c), lambda i: (0, 0)),
        ],
        out_shape=[
            jax.ShapeDtypeStruct((B, c), jnp.float32),
            jax.ShapeDtypeStruct((B, c), jnp.float32),
        ],
    )(bidx3_tc, features)

    rb3 = 16000
    assert n % rb3 == 0
    nb = n // rb3
    bidx3 = batch_idx.reshape(nb, 1, rb3)
    out = pl.pallas_call(
        _p3g,
        grid=(nb,),
        in_specs=[
            pl.BlockSpec((1, 1, rb3), lambda i: (i, 0, 0)),
            pl.BlockSpec((rb3, c), lambda i: (i, 0)),
            pl.BlockSpec((B, c), lambda i: (0, 0)),
            pl.BlockSpec((B, c), lambda i: (0, 0)),
            pl.BlockSpec((NW, B, c), lambda i: (0, 0, 0)),
            pl.BlockSpec((NW, L), lambda i: (0, 0)),
            pl.BlockSpec((1, 3), lambda i: (0, 0)),
        ],
        out_specs=pl.BlockSpec((rb3, c), lambda i: (i, 0)),
        out_shape=jax.ShapeDtypeStruct((n, c), jnp.float32),
        scratch_shapes=[pltpu.VMEM((B, c), jnp.float32)],
    )(bidx3, features, tsums, tcnt, psum.reshape(NW, B, c),
      pcnt.reshape(NW, L), W.reshape(1, 3))
    return out


# speedup vs baseline: 1.0221x; 1.0106x over previous
import functools

import jax
import jax.numpy as jnp
from jax import lax
from jax.experimental import pallas as pl
from jax.experimental.pallas import tpu as pltpu
from jax.experimental.pallas import tpu_sc as plsc

B = 16
L = 16
NC, NS = 2, 16
NW = NC * NS


def _lane(vec, s, iota):
    return jnp.sum(jnp.where(iota == s, vec, 0))


def _seg_bounds(bidx_v, r, iota):
    lo = jnp.zeros((L,), jnp.int32)
    hi = jnp.full((L,), r, jnp.int32)
    steps = max(1, (r).bit_length())

    def body(_, carry):
        lo, hi = carry
        active = lo < hi
        mid = lax.div(lo + hi, 2)
        vals = plsc.load_gather(bidx_v, [jnp.minimum(mid, r - 1)])
        pred = vals < iota
        lo = jnp.where(active & pred, mid + 1, lo)
        hi = jnp.where(active & jnp.logical_not(pred), mid, hi)
        return lo, hi

    lo, hi = lax.fori_loop(0, steps, body, (lo, hi))
    starts = [_lane(lo, s, iota) for s in range(B)]
    ends = starts[1:] + [jnp.int32(r)]
    return starts, ends


def _make_sums_kernel(c, r, ch):
    nch = r // ch
    mesh = plsc.VectorSubcoreMesh(core_axis_name="c", subcore_axis_name="s")

    @functools.partial(
        pl.kernel,
        out_type=[
            jax.ShapeDtypeStruct((NW, B * c), jnp.float32),
            jax.ShapeDtypeStruct((NW * L,), jnp.float32),
        ],
        mesh=mesh,
        compiler_params=pltpu.CompilerParams(needs_layout_passes=False),
        scratch_types=[
            pltpu.VMEM((r,), jnp.int32),
            pltpu.VMEM((ch * c,), jnp.float32),
            pltpu.VMEM((ch * c,), jnp.float32),
            pltpu.VMEM((B * c,), jnp.float32),
            pltpu.VMEM((L,), jnp.float32),
            pltpu.SemaphoreType.DMA,
            pltpu.SemaphoreType.DMA,
            pltpu.SemaphoreType.DMA,
        ],
    )
    def sums_kernel(feat_hbm, bidx_hbm, psum_hbm, pcnt_hbm,
                    bidx_v, f0, f1, acc_v, cnt_v, semi, sem0, sem1):
        wid = lax.axis_index("c") * NS + lax.axis_index("s")
        base = wid * r
        iota = lax.iota(jnp.int32, L)
        zero16 = jnp.zeros((L,), jnp.float32)

        fbs = (f0, f1)
        sems = (sem0, sem1)
        pltpu.async_copy(bidx_hbm.at[pl.ds(base, r)], bidx_v, semi)
        for b in range(2):
            pltpu.async_copy(
                feat_hbm.at[pl.ds((base + b * ch) * c, ch * c)], fbs[b],
                sems[b])

        for q in range(B * c // L):
            acc_v[pl.ds(q * L, L)] = zero16

        pltpu.make_async_copy(bidx_hbm.at[pl.ds(base, r)], bidx_v, semi).wait()
        starts, ends = _seg_bounds(bidx_v, r, iota)

        cnt_f = zero16
        for s in range(B):
            cnt_f = jnp.where(iota == s,
                              (ends[s] - starts[s]).astype(jnp.float32), cnt_f)
        cnt_v[...] = cnt_f

        def chunk_pair(p, carry):
            for b in range(2):
                j = p * 2 + b
                cb = j * ch
                fb = fbs[b]
                semb = sems[b]
                pltpu.make_async_copy(
                    feat_hbm.at[pl.ds((base + cb) * c, ch * c)], fb, semb
                ).wait()
                for s in range(B):
                    lo = jnp.maximum(starts[s] - cb, 0)
                    hi = jnp.minimum(ends[s] - cb, ch)

                    @pl.when(hi > lo)
                    def _(s=s, lo=lo, hi=hi, fb=fb):
                        acc8 = tuple(
                            acc_v[pl.ds(s * c + k * L, L)] for k in range(c // L)
                        )

                        @plsc.parallel_loop(lo, hi, unroll=2, carry=acc8)
                        def a8(rr, a8, fb=fb):
                            return tuple(
                                a + fb[pl.ds(rr * c + k * L, L)]
                                for k, a in enumerate(a8)
                            )

                        for k in range(c // L):
                            acc_v[pl.ds(s * c + k * L, L)] = a8[k]

                nxt = j + 2

                @pl.when(nxt < nch)
                def _(nxt=nxt, fb=fb, semb=semb):
                    pltpu.async_copy(
                        feat_hbm.at[pl.ds((base + nxt * ch) * c, ch * c)],
                        fb, semb)
            return carry

        lax.fori_loop(0, nch // 2, chunk_pair, 0)

        pltpu.sync_copy(acc_v, psum_hbm.at[wid])
        pltpu.sync_copy(cnt_v, pcnt_hbm.at[pl.ds(wid * L, L)])

    return sums_kernel


def _p1(bidx_ref, feat_ref, sums_ref, cnt_ref):
    i = pl.program_id(0)
    rb = feat_ref.shape[0]
    b = bidx_ref[0, 0, :]
    onehot = (b[:, None] == jax.lax.broadcasted_iota(jnp.int32, (rb, B), 1)
              ).astype(jnp.float32)
    part = jax.lax.dot_general(onehot, feat_ref[...],
                               (((0,), (0,)), ((), ())),
                               preferred_element_type=jnp.float32)
    pcnt = jnp.sum(onehot, axis=0)[:, None]

    @pl.when(i == 0)
    def _():
        sums_ref[...] = jnp.zeros_like(sums_ref)
        cnt_ref[...] = jnp.zeros_like(cnt_ref)

    sums_ref[...] += part
    cnt_ref[...] += jnp.broadcast_to(pcnt, cnt_ref.shape)


def _p3g(bidx_ref, feat_ref, tsums_ref, tcnt_ref, psum_ref, pcnt_ref, w_ref,
         out_ref, gates_ref):
    i = pl.program_id(0)
    rb = feat_ref.shape[0]

    @pl.when(i == 0)
    def _():
        sums = tsums_ref[...] + jnp.sum(psum_ref[...], axis=0)
        cnt = tcnt_ref[...] + jnp.sum(pcnt_ref[...], axis=0)[:, None]
        m = sums / jnp.maximum(cnt, 1.0)
        w0 = w_ref[0, 0]
        w1 = w_ref[0, 1]
        w2 = w_ref[0, 2]
        zero = jnp.zeros((m.shape[0], 1), jnp.float32)
        left = jnp.concatenate([zero, m[:, :-1]], axis=1)
        right = jnp.concatenate([m[:, 1:], zero], axis=1)
        y = w0 * left + w1 * m + w2 * right
        gates_ref[...] = jax.nn.sigmoid(y)

    b = bidx_ref[0, 0, :]
    onehot = (b[:, None] == jax.lax.broadcasted_iota(jnp.int32, (rb, B), 1)
              ).astype(jnp.float32)
    g = jax.lax.dot_general(onehot, gates_ref[...],
                            (((1,), (0,)), ((), ())),
                            preferred_element_type=jnp.float32)
    out_ref[...] = feat_ref[...] * g


N_SC_FRAC_NUM, N_SC_FRAC_DEN = 3, 10


def kernel(features, batch_idx, W):
    n, c = features.shape
    assert c % L == 0
    rb = 16000
    grain = NW * 1000
    n_sc = max(grain, (n * N_SC_FRAC_NUM // N_SC_FRAC_DEN) // grain * grain)
    r = n_sc // NW
    ch_a = r // 10
    assert r % ch_a == 0 and (r // ch_a) % 2 == 0 and r % 8 == 0, (r, ch_a)
    n_tc = n - n_sc

    featflat = features.reshape(-1)

    psum, pcnt = _make_sums_kernel(c, r, ch_a)(featflat, batch_idx)

    assert n_tc % rb == 0 and n_sc % rb == 0
    nb_tc = n_tc // rb
    blk0 = n_sc // rb
    bidx3_tc = batch_idx[n_sc:].reshape(nb_tc, 1, rb)
    tsums, tcnt = pl.pallas_call(
        _p1,
        grid=(nb_tc,),
        in_specs=[
            pl.BlockSpec((1, 1, rb), lambda i: (i, 0, 0)),
            pl.BlockSpec((rb, c), lambda i: (i + blk0, 0)),
        ],
        out_specs=[
            pl.BlockSpec((B, c), lambda i: (0, 0)),
            pl.BlockSpec((B, c), lambda i: (0, 0)),
        ],
        out_shape=[
            jax.ShapeDtypeStruct((B, c), jnp.float32),
            jax.ShapeDtypeStruct((B, c), jnp.float32),
        ],
    )(bidx3_tc, features)

    rb3 = 16000
    assert n % rb3 == 0
    nb = n // rb3
    bidx3 = batch_idx.reshape(nb, 1, rb3)
    out = pl.pallas_call(
        _p3g,
        grid=(nb,),
        in_specs=[
            pl.BlockSpec((1, 1, rb3), lambda i: (i, 0, 0)),
            pl.BlockSpec((rb3, c), lambda i: (i, 0)),
            pl.BlockSpec((B, c), lambda i: (0, 0)),
            pl.BlockSpec((B, c), lambda i: (0, 0)),
            pl.BlockSpec((NW, B, c), lambda i: (0, 0, 0)),
            pl.BlockSpec((NW, L), lambda i: (0, 0)),
            pl.BlockSpec((1, 3), lambda i: (0, 0)),
        ],
        out_specs=pl.BlockSpec((rb3, c), lambda i: (i, 0)),
        out_shape=jax.ShapeDtypeStruct((n, c), jnp.float32),
        scratch_shapes=[pltpu.VMEM((B, c), jnp.float32)],
    )(bidx3, features, tsums, tcnt, psum.reshape(NW, B, c),
      pcnt.reshape(NW, L), W.reshape(1, 3))
    return out


# speedup vs baseline: 1.0399x; 1.0174x over previous
import functools

import jax
import jax.numpy as jnp
from jax import lax
from jax.experimental import pallas as pl
from jax.experimental.pallas import tpu as pltpu
from jax.experimental.pallas import tpu_sc as plsc

B = 16
L = 16
NC, NS = 2, 16
NW = NC * NS


def _lane(vec, s, iota):
    return jnp.sum(jnp.where(iota == s, vec, 0))


def _seg_bounds(bidx_v, r, iota):
    lo = jnp.zeros((L,), jnp.int32)
    hi = jnp.full((L,), r, jnp.int32)
    steps = max(1, (r).bit_length())

    def body(_, carry):
        lo, hi = carry
        active = lo < hi
        mid = lax.div(lo + hi, 2)
        vals = plsc.load_gather(bidx_v, [jnp.minimum(mid, r - 1)])
        pred = vals < iota
        lo = jnp.where(active & pred, mid + 1, lo)
        hi = jnp.where(active & jnp.logical_not(pred), mid, hi)
        return lo, hi

    lo, hi = lax.fori_loop(0, steps, body, (lo, hi))
    starts = [_lane(lo, s, iota) for s in range(B)]
    ends = starts[1:] + [jnp.int32(r)]
    return starts, ends


def _make_sums_kernel(c, r, ch):
    nch = r // ch
    mesh = plsc.VectorSubcoreMesh(core_axis_name="c", subcore_axis_name="s")

    @functools.partial(
        pl.kernel,
        out_type=[
            jax.ShapeDtypeStruct((NW, B * c), jnp.float32),
            jax.ShapeDtypeStruct((NW * L,), jnp.float32),
        ],
        mesh=mesh,
        compiler_params=pltpu.CompilerParams(needs_layout_passes=False),
        scratch_types=[
            pltpu.VMEM((r,), jnp.int32),
            pltpu.VMEM((ch * c,), jnp.float32),
            pltpu.VMEM((ch * c,), jnp.float32),
            pltpu.VMEM((B * c,), jnp.float32),
            pltpu.VMEM((L,), jnp.float32),
            pltpu.SemaphoreType.DMA,
            pltpu.SemaphoreType.DMA,
            pltpu.SemaphoreType.DMA,
        ],
    )
    def sums_kernel(feat_hbm, bidx_hbm, psum_hbm, pcnt_hbm,
                    bidx_v, f0, f1, acc_v, cnt_v, semi, sem0, sem1):
        wid = lax.axis_index("c") * NS + lax.axis_index("s")
        base = wid * r
        iota = lax.iota(jnp.int32, L)
        zero16 = jnp.zeros((L,), jnp.float32)

        fbs = (f0, f1)
        sems = (sem0, sem1)
        pltpu.async_copy(bidx_hbm.at[pl.ds(base, r)], bidx_v, semi)
        for b in range(2):
            pltpu.async_copy(
                feat_hbm.at[pl.ds((base + b * ch) * c, ch * c)], fbs[b],
                sems[b])

        for q in range(B * c // L):
            acc_v[pl.ds(q * L, L)] = zero16

        pltpu.make_async_copy(bidx_hbm.at[pl.ds(base, r)], bidx_v, semi).wait()
        starts, ends = _seg_bounds(bidx_v, r, iota)

        cnt_f = zero16
        for s in range(B):
            cnt_f = jnp.where(iota == s,
                              (ends[s] - starts[s]).astype(jnp.float32), cnt_f)
        cnt_v[...] = cnt_f

        def chunk_pair(p, carry):
            for b in range(2):
                j = p * 2 + b
                cb = j * ch
                fb = fbs[b]
                semb = sems[b]
                pltpu.make_async_copy(
                    feat_hbm.at[pl.ds((base + cb) * c, ch * c)], fb, semb
                ).wait()
                for s in range(B):
                    lo = jnp.maximum(starts[s] - cb, 0)
                    hi = jnp.minimum(ends[s] - cb, ch)

                    @pl.when(hi > lo)
                    def _(s=s, lo=lo, hi=hi, fb=fb):
                        acc8 = tuple(
                            acc_v[pl.ds(s * c + k * L, L)] for k in range(c // L)
                        )

                        @plsc.parallel_loop(lo, hi, unroll=2, carry=acc8)
                        def a8(rr, a8, fb=fb):
                            return tuple(
                                a + fb[pl.ds(rr * c + k * L, L)]
                                for k, a in enumerate(a8)
                            )

                        for k in range(c // L):
                            acc_v[pl.ds(s * c + k * L, L)] = a8[k]

                nxt = j + 2

                @pl.when(nxt < nch)
                def _(nxt=nxt, fb=fb, semb=semb):
                    pltpu.async_copy(
                        feat_hbm.at[pl.ds((base + nxt * ch) * c, ch * c)],
                        fb, semb)
            return carry

        lax.fori_loop(0, nch // 2, chunk_pair, 0)

        pltpu.sync_copy(acc_v, psum_hbm.at[wid])
        pltpu.sync_copy(cnt_v, pcnt_hbm.at[pl.ds(wid * L, L)])

    return sums_kernel


def _p1(bidx_ref, feat_ref, sums_ref, cnt_ref):
    i = pl.program_id(0)
    rb = feat_ref.shape[0]
    b = bidx_ref[0, 0, :]
    onehot = (b[:, None] == jax.lax.broadcasted_iota(jnp.int32, (rb, B), 1)
              ).astype(jnp.float32)
    part = jax.lax.dot_general(onehot, feat_ref[...],
                               (((0,), (0,)), ((), ())),
                               preferred_element_type=jnp.float32)
    pcnt = jnp.sum(onehot, axis=0)[:, None]

    @pl.when(i == 0)
    def _():
        sums_ref[...] = jnp.zeros_like(sums_ref)
        cnt_ref[...] = jnp.zeros_like(cnt_ref)

    sums_ref[...] += part
    cnt_ref[...] += jnp.broadcast_to(pcnt, cnt_ref.shape)


def _p3g(bidx_ref, feat_ref, tsums_ref, tcnt_ref, psum_ref, pcnt_ref, w_ref,
         out_ref, gates_ref):
    i = pl.program_id(0)
    rb = feat_ref.shape[0]

    @pl.when(i == 0)
    def _():
        sums = tsums_ref[...] + jnp.sum(psum_ref[...], axis=0)
        cnt = tcnt_ref[...] + jnp.sum(pcnt_ref[...], axis=0)[:, None]
        m = sums / jnp.maximum(cnt, 1.0)
        w0 = w_ref[0, 0]
        w1 = w_ref[0, 1]
        w2 = w_ref[0, 2]
        zero = jnp.zeros((m.shape[0], 1), jnp.float32)
        left = jnp.concatenate([zero, m[:, :-1]], axis=1)
        right = jnp.concatenate([m[:, 1:], zero], axis=1)
        y = w0 * left + w1 * m + w2 * right
        gates_ref[...] = jax.nn.sigmoid(y)

    b = bidx_ref[0, 0, :]
    onehot = (b[:, None] == jax.lax.broadcasted_iota(jnp.int32, (rb, B), 1)
              ).astype(jnp.float32)
    g = jax.lax.dot_general(onehot, gates_ref[...],
                            (((1,), (0,)), ((), ())),
                            preferred_element_type=jnp.float32)
    out_ref[...] = feat_ref[...] * g


N_SC_FRAC_NUM, N_SC_FRAC_DEN = 2, 5


def kernel(features, batch_idx, W):
    n, c = features.shape
    assert c % L == 0
    rb = 16000
    grain = NW * 1000
    n_sc = max(grain, (n * N_SC_FRAC_NUM // N_SC_FRAC_DEN) // grain * grain)
    r = n_sc // NW
    ch_a = r // 10
    assert r % ch_a == 0 and (r // ch_a) % 2 == 0 and r % 8 == 0, (r, ch_a)
    n_tc = n - n_sc

    featflat = features.reshape(-1)

    psum, pcnt = _make_sums_kernel(c, r, ch_a)(featflat, batch_idx)

    assert n_tc % rb == 0 and n_sc % rb == 0
    nb_tc = n_tc // rb
    blk0 = n_sc // rb
    bidx3_full = batch_idx.reshape(n // rb, 1, rb)
    tsums, tcnt = pl.pallas_call(
        _p1,
        grid=(nb_tc,),
        in_specs=[
            pl.BlockSpec((1, 1, rb), lambda i: (i + blk0, 0, 0)),
            pl.BlockSpec((rb, c), lambda i: (i + blk0, 0)),
        ],
        out_specs=[
            pl.BlockSpec((B, c), lambda i: (0, 0)),
            pl.BlockSpec((B, c), lambda i: (0, 0)),
        ],
        out_shape=[
            jax.ShapeDtypeStruct((B, c), jnp.float32),
            jax.ShapeDtypeStruct((B, c), jnp.float32),
        ],
    )(bidx3_full, features)

    rb3 = 16000
    assert n % rb3 == 0
    nb = n // rb3
    bidx3 = batch_idx.reshape(nb, 1, rb3)
    out = pl.pallas_call(
        _p3g,
        grid=(nb,),
        in_specs=[
            pl.BlockSpec((1, 1, rb3), lambda i: (i, 0, 0)),
            pl.BlockSpec((rb3, c), lambda i: (i, 0)),
            pl.BlockSpec((B, c), lambda i: (0, 0)),
            pl.BlockSpec((B, c), lambda i: (0, 0)),
            pl.BlockSpec((NW, B, c), lambda i: (0, 0, 0)),
            pl.BlockSpec((NW, L), lambda i: (0, 0)),
            pl.BlockSpec((1, 3), lambda i: (0, 0)),
        ],
        out_specs=pl.BlockSpec((rb3, c), lambda i: (i, 0)),
        out_shape=jax.ShapeDtypeStruct((n, c), jnp.float32),
        scratch_shapes=[pltpu.VMEM((B, c), jnp.float32)],
    )(bidx3, features, tsums, tcnt, psum.reshape(NW, B, c),
      pcnt.reshape(NW, L), W.reshape(1, 3))
    return out
